# baseline (speedup 1.0000x reference)
import jax
import jax.numpy as jnp
from jax.experimental import pallas as pl

N = 25
C = 64
H = 8
DH = 8
B = 256
CLASSES = 60
FB = 256


def _lrelu(x):
    return jnp.where(x >= 0, x, 0.2 * x)


def _layer_body(x, W_ref, As_ref, Ad_ref, LC_ref, G_ref, GT_ref,
                Wq_ref, Wk_ref, Wv_ref, Wo_ref, W1_ref, W2_ref,
                y_ref, a_ref, st_ref):
    f32 = jnp.float32
    x2 = x.reshape(N * FB, C)
    h2 = jnp.dot(x2, W_ref[...], preferred_element_type=f32)
    h3 = h2.reshape(N, FB, C)
    ls = jnp.dot(h2, As_ref[...], preferred_element_type=f32).reshape(N, FB, H)
    ld = jnp.dot(h2, Ad_ref[...], preferred_element_type=f32).reshape(N, FB, H)
    outs = []
    for n in range(N):
        s_n = _lrelu(ls + ld[n][None])
        w_n = jnp.exp(s_n + LC_ref[n][:, None, :])
        den = jnp.maximum(w_n.sum(axis=0, keepdims=True), 1e-30)
        attn = (w_n / den).reshape(N * FB, H)
        attn_e = jnp.dot(attn, GT_ref[...],
                         preferred_element_type=f32).reshape(N, FB, C)
        outs.append((attn_e * h3).sum(axis=0)[None])
    a = jnp.maximum(jnp.concatenate(outs, axis=0), 0.0)
    a_ref[...] = a

    a2 = a.reshape(N * FB, C)
    q3 = jnp.dot(a2, Wq_ref[...], preferred_element_type=f32).reshape(N, FB, C)
    k2 = jnp.dot(a2, Wk_ref[...], preferred_element_type=f32)
    k3 = k2.reshape(N, FB, C)
    v3 = jnp.dot(a2, Wv_ref[...], preferred_element_type=f32).reshape(N, FB, C)
    scale = 1.0 / jnp.sqrt(jnp.float32(DH))
    outs = []
    for n in range(N):
        p = (k3 * q3[n][None]).reshape(N * FB, C)
        s_n = jnp.dot(p, G_ref[...],
                      preferred_element_type=f32).reshape(N, FB, H) * scale
        s_n = s_n - s_n.max(axis=0, keepdims=True)
        e_n = jnp.exp(s_n)
        attn = (e_n / e_n.sum(axis=0, keepdims=True)).reshape(N * FB, H)
        attn_e = jnp.dot(attn, GT_ref[...],
                         preferred_element_type=f32).reshape(N, FB, C)
        outs.append((attn_e * v3).sum(axis=0)[None])
    o = jnp.concatenate(outs, axis=0).reshape(N * FB, C)
    o = jnp.dot(o, Wo_ref[...], preferred_element_type=f32)
    y2 = jnp.dot(jnp.maximum(jnp.dot(o, W1_ref[...],
                                     preferred_element_type=f32), 0.0),
                 W2_ref[...], preferred_element_type=f32)
    y = y2.reshape(N, FB, C)
    y_ref[...] = y

    @pl.when(pl.program_id(0) == 0)
    def _():
        st_ref[...] = jnp.zeros(st_ref.shape, st_ref.dtype)

    st_ref[0] += y.sum(axis=1)
    st_ref[1] += (y * y).sum(axis=1)


def _layer_first(x_ref, *refs):
    _layer_body(x_ref[...], *refs)


def _layer_norm(p0_ref, p1_ref, stats_ref, *refs):
    mu = stats_ref[0]
    rstd = stats_ref[1]
    x = jnp.maximum((p0_ref[...] - mu[:, None, :]) * rstd[:, None, :]
                    + p1_ref[...], 0.0)
    _layer_body(x, *refs)


def _pool_kernel(p0_ref, p1_ref, stats_ref, Wg_ref, u_ref, OT_ref,
                 num_ref, den_ref):
    f32 = jnp.float32
    mu = stats_ref[0]
    rstd = stats_ref[1]
    x = jnp.maximum((p0_ref[...] - mu[:, None, :]) * rstd[:, None, :]
                    + p1_ref[...], 0.0)
    x2 = x.reshape(N * FB, C)
    g = jnp.tanh(jnp.dot(x2, Wg_ref[...], preferred_element_type=f32))
    e = jnp.exp(jnp.dot(g, u_ref[...], preferred_element_type=f32))
    w3 = (x2 * e).reshape(N, FB, C)
    e3 = e.reshape(N, FB, 1)
    ot = OT_ref[...]

    @pl.when(pl.program_id(0) == 0)
    def _():
        num_ref[...] = jnp.zeros(num_ref.shape, num_ref.dtype)
        den_ref[...] = jnp.zeros(den_ref.shape, den_ref.dtype)

    for n in range(N):
        num_ref[n] += jnp.dot(ot, w3[n], preferred_element_type=f32)
        den_ref[n] += jnp.dot(ot, e3[n], preferred_element_type=f32)


def _final_kernel(num_ref, den_ref, Wf_ref, bf_ref, out_ref):
    cols = []
    for n in range(N):
        d = jnp.maximum(den_ref[n], 1e-30)
        cols.append(jnp.maximum(num_ref[n] / d, 0.0))
    flat = jnp.concatenate(cols, axis=1)
    out_ref[...] = (jnp.dot(flat, Wf_ref[...],
                            preferred_element_type=jnp.float32)
                    + bf_ref[...])


def _full(shape):
    return pl.BlockSpec(shape, lambda i: tuple(0 for _ in shape))


def kernel(t, adj, bi, params):
    F = t.shape[0]
    G = F // FB
    f32 = jnp.float32
    Ss = jax.nn.one_hot(adj[0], N, dtype=f32)
    Sd = jax.nn.one_hot(adj[1], N, dtype=f32)
    cnt = Sd.T @ Ss
    lcnt = jnp.where(cnt > 0, jnp.log(jnp.maximum(cnt, 1e-30)), -1e30)
    LC = jnp.broadcast_to(lcnt[:, :, None], (N, N, H))
    OT = jax.nn.one_hot(bi, B, dtype=f32).T
    Gm = jnp.repeat(jnp.eye(H, dtype=f32), DH, axis=0)
    GT = Gm.T
    eyeH = jnp.eye(H, dtype=f32)

    def amat(a):
        return (a[:, :, None] * eyeH[:, None, :]).reshape(C, H)

    x0 = jnp.transpose(t, (1, 0, 2))

    tile = pl.BlockSpec((N, FB, C), lambda i: (0, i, 0))
    w_specs = [_full((C, C)), _full((C, H)), _full((C, H)),
               _full((N, N, H)), _full((C, H)), _full((H, C))] + \
              [_full((C, C))] * 6
    out_shapes = [jax.ShapeDtypeStruct((N, F, C), f32),
                  jax.ShapeDtypeStruct((N, F, C), f32),
                  jax.ShapeDtypeStruct((2, N, C), f32)]
    out_specs = [tile, tile, _full((2, N, C))]

    def wargs(p):
        return [p['W'], amat(p['a_src']), amat(p['a_dst']), LC, Gm, GT,
                p['Wq'], p['Wk'], p['Wv'], p['Wo'], p['W1'], p['W2']]

    def stats_of(st):
        cntf = jnp.float32(F * C)
        mu = st[0].sum(-1) / cntf
        var = st[1].sum(-1) / cntf - mu * mu
        rstd = 1.0 / jnp.sqrt(var + 1e-5)
        return jnp.stack([jnp.broadcast_to(mu[:, None], (N, C)),
                          jnp.broadcast_to(rstd[:, None], (N, C))])

    y, a, st = pl.pallas_call(
        _layer_first, grid=(G,),
        in_specs=[tile] + w_specs,
        out_specs=out_specs, out_shape=out_shapes,
    )(x0, *wargs(params['layer0']))

    for i in (1, 2):
        y, a, st = pl.pallas_call(
            _layer_norm, grid=(G,),
            in_specs=[tile, tile, _full((2, N, C))] + w_specs,
            out_specs=out_specs, out_shape=out_shapes,
        )(y, a, stats_of(st), *wargs(params['layer%d' % i]))

    num, den = pl.pallas_call(
        _pool_kernel, grid=(G,),
        in_specs=[tile, tile, _full((2, N, C)), _full((C, C)),
                  _full((C, 1)), pl.BlockSpec((B, FB), lambda i: (0, i))],
        out_specs=[_full((N, B, C)), _full((N, B, 1))],
        out_shape=[jax.ShapeDtypeStruct((N, B, C), f32),
                   jax.ShapeDtypeStruct((N, B, 1), f32)],
    )(y, a, stats_of(st), params['Wg'], params['u'].reshape(C, 1), OT)

    out = pl.pallas_call(
        _final_kernel, grid=(1,),
        in_specs=[_full((N, B, C)), _full((N, B, 1)),
                  _full((N * C, CLASSES)), _full((1, CLASSES))],
        out_specs=_full((B, CLASSES)),
        out_shape=jax.ShapeDtypeStruct((B, CLASSES), f32),
    )(num, den, params['Wf'], params['bf'].reshape(1, CLASSES))
    return out
